# baseline (speedup 1.0000x reference)
import jax
import jax.numpy as jnp
from jax.experimental import pallas as pl
from jax.experimental.pallas import tpu as pltpu

_LANES = 128


def _fc_body(x_ref, w_ref, b_ref, o_ref):
    xb = x_ref[...].astype(jnp.bfloat16)
    acc = jnp.dot(xb, w_ref[...], preferred_element_type=jnp.float32)
    o_ref[...] = acc + b_ref[0, 0]


def kernel(x, weight, bias):
    B, F = x.shape
    chunk = _LANES * F

    b_pad = ((B + _LANES - 1) // _LANES) * _LANES
    if b_pad != B:
        x = jnp.pad(x, ((0, b_pad - B), (0, 0)))
    rows = b_pad // _LANES

    x_rows = x.reshape(rows, chunk)

    w_flat = weight.reshape(F).astype(jnp.float32)
    gather = jnp.repeat(jnp.eye(_LANES, dtype=jnp.float32), F, axis=0)
    w_big = (gather * jnp.tile(w_flat, _LANES)[:, None]).astype(jnp.bfloat16)
    b_smem = bias.reshape(1, 1).astype(jnp.float32)

    block_rows = min(512, rows)
    block_rows = max(8, (block_rows // 8) * 8)
    grid = (pl.cdiv(rows, block_rows),)

    out = pl.pallas_call(
        _fc_body,
        out_shape=jax.ShapeDtypeStruct((rows, _LANES), jnp.float32),
        grid=grid,
        in_specs=[
            pl.BlockSpec((block_rows, chunk), lambda i: (i, 0)),
            pl.BlockSpec((chunk, _LANES), lambda i: (0, 0)),
            pl.BlockSpec(memory_space=pltpu.MemorySpace.SMEM),
        ],
        out_specs=pl.BlockSpec((block_rows, _LANES), lambda i: (i, 0)),
        compiler_params=pltpu.CompilerParams(
            dimension_semantics=("parallel",),
        ),
        cost_estimate=pl.CostEstimate(
            flops=6 * b_pad, transcendentals=0, bytes_accessed=16 * b_pad),
    )(x_rows, w_big, b_smem)

    y = out.reshape(b_pad, 1)
    if b_pad != B:
        y = y[:B]
    return y


# speedup vs baseline: 19.3773x; 19.3773x over previous
import jax
import jax.numpy as jnp
from jax.experimental import pallas as pl
from jax.experimental.pallas import tpu as pltpu

_LANES = 128
_ROWS_PER_BLOCK = 64


def _make_fc_body(rpb):
    def _fc_body(xt_ref, wb_ref, o_ref):
        w0 = wb_ref[0, 0]
        w1 = wb_ref[0, 1]
        w2 = wb_ref[0, 2]
        b = wb_ref[0, 3]
        for r in range(rpb):
            s = slice(r * _LANES, (r + 1) * _LANES)
            o_ref[r : r + 1, :] = (
                w0 * xt_ref[0:1, s] + w1 * xt_ref[1:2, s] + w2 * xt_ref[2:3, s] + b
            )

    return _fc_body


def kernel(x, weight, bias):
    B, F = x.shape
    assert F == 3

    b_pad = ((B + _LANES - 1) // _LANES) * _LANES
    if b_pad != B:
        x = jnp.pad(x, ((0, b_pad - B), (0, 0)))
    rows = b_pad // _LANES

    xt = x.T

    wb = jnp.concatenate(
        [weight.reshape(F).astype(jnp.float32), bias.astype(jnp.float32)]
    ).reshape(1, 4)

    rpb = min(_ROWS_PER_BLOCK, rows)
    grid = (pl.cdiv(rows, rpb),)

    out = pl.pallas_call(
        _make_fc_body(rpb),
        out_shape=jax.ShapeDtypeStruct((rows, _LANES), jnp.float32),
        grid=grid,
        in_specs=[
            pl.BlockSpec((3, rpb * _LANES), lambda i: (0, i)),
            pl.BlockSpec(memory_space=pltpu.MemorySpace.SMEM),
        ],
        out_specs=pl.BlockSpec((rpb, _LANES), lambda i: (i, 0)),
        compiler_params=pltpu.CompilerParams(
            dimension_semantics=("parallel",),
        ),
        cost_estimate=pl.CostEstimate(
            flops=6 * b_pad, transcendentals=0, bytes_accessed=16 * b_pad),
    )(xt, wb)

    y = out.reshape(b_pad, 1)
    if b_pad != B:
        y = y[:B]
    return y


# speedup vs baseline: 56.5194x; 2.9168x over previous
import jax
import jax.numpy as jnp
from jax.experimental import pallas as pl
from jax.experimental.pallas import tpu as pltpu

_LANES = 128
_ROWS_PER_BLOCK = 256


def _make_fc_body(rpb):
    def _fc_body(xt_ref, wb_ref, o_ref):
        w0 = wb_ref[0, 0]
        w1 = wb_ref[0, 1]
        w2 = wb_ref[0, 2]
        b = wb_ref[0, 3]
        for r in range(rpb):
            s = slice(r * _LANES, (r + 1) * _LANES)
            o_ref[r : r + 1, :] = (
                w0 * xt_ref[0:1, s] + w1 * xt_ref[1:2, s] + w2 * xt_ref[2:3, s] + b
            )

    return _fc_body


def kernel(x, weight, bias):
    B, F = x.shape
    assert F == 3

    b_pad = ((B + _LANES - 1) // _LANES) * _LANES
    if b_pad != B:
        x = jnp.pad(x, ((0, b_pad - B), (0, 0)))
    rows = b_pad // _LANES

    xt = x.T

    wb = jnp.concatenate(
        [weight.reshape(F).astype(jnp.float32), bias.astype(jnp.float32)]
    ).reshape(1, 4)

    rpb = min(_ROWS_PER_BLOCK, rows)
    grid = (pl.cdiv(rows, rpb),)

    out = pl.pallas_call(
        _make_fc_body(rpb),
        out_shape=jax.ShapeDtypeStruct((rows, _LANES), jnp.float32),
        grid=grid,
        in_specs=[
            pl.BlockSpec((3, rpb * _LANES), lambda i: (0, i)),
            pl.BlockSpec(memory_space=pltpu.MemorySpace.SMEM),
        ],
        out_specs=pl.BlockSpec((rpb, _LANES), lambda i: (i, 0)),
        compiler_params=pltpu.CompilerParams(
            dimension_semantics=("parallel",),
        ),
        cost_estimate=pl.CostEstimate(
            flops=6 * b_pad, transcendentals=0, bytes_accessed=16 * b_pad),
    )(xt, wb)

    y = out.reshape(b_pad, 1)
    if b_pad != B:
        y = y[:B]
    return y


# speedup vs baseline: 82.2136x; 1.4546x over previous
import jax
import jax.numpy as jnp
from jax.experimental import pallas as pl
from jax.experimental.pallas import tpu as pltpu

_LANES = 128
_ROWS_PER_BLOCK = 512


def _make_fc_body(rpb):
    def _fc_body(xt_ref, wb_ref, o_ref):
        w0 = wb_ref[0, 0]
        w1 = wb_ref[0, 1]
        w2 = wb_ref[0, 2]
        b = wb_ref[0, 3]
        for r in range(rpb):
            s = slice(r * _LANES, (r + 1) * _LANES)
            o_ref[r : r + 1, :] = (
                w0 * xt_ref[0:1, s] + w1 * xt_ref[1:2, s] + w2 * xt_ref[2:3, s] + b
            )

    return _fc_body


def kernel(x, weight, bias):
    B, F = x.shape
    assert F == 3

    b_pad = ((B + _LANES - 1) // _LANES) * _LANES
    if b_pad != B:
        x = jnp.pad(x, ((0, b_pad - B), (0, 0)))
    rows = b_pad // _LANES

    xt = x.T

    wb = jnp.concatenate(
        [weight.reshape(F).astype(jnp.float32), bias.astype(jnp.float32)]
    ).reshape(1, 4)

    rpb = min(_ROWS_PER_BLOCK, rows)
    grid = (pl.cdiv(rows, rpb),)

    out = pl.pallas_call(
        _make_fc_body(rpb),
        out_shape=jax.ShapeDtypeStruct((rows, _LANES), jnp.float32),
        grid=grid,
        in_specs=[
            pl.BlockSpec((3, rpb * _LANES), lambda i: (0, i)),
            pl.BlockSpec(memory_space=pltpu.MemorySpace.SMEM),
        ],
        out_specs=pl.BlockSpec((rpb, _LANES), lambda i: (i, 0)),
        compiler_params=pltpu.CompilerParams(
            dimension_semantics=("parallel",),
        ),
        cost_estimate=pl.CostEstimate(
            flops=6 * b_pad, transcendentals=0, bytes_accessed=16 * b_pad),
    )(xt, wb)

    y = out.reshape(b_pad, 1)
    if b_pad != B:
        y = y[:B]
    return y


# speedup vs baseline: 106.1725x; 1.2914x over previous
import jax
import jax.numpy as jnp
from jax.experimental import pallas as pl
from jax.experimental.pallas import tpu as pltpu

_LANES = 128
_ROWS_PER_BLOCK = 1024


def _make_fc_body(rpb):
    def _fc_body(xt_ref, wb_ref, o_ref):
        w0 = wb_ref[0, 0]
        w1 = wb_ref[0, 1]
        w2 = wb_ref[0, 2]
        b = wb_ref[0, 3]
        for r in range(rpb):
            s = slice(r * _LANES, (r + 1) * _LANES)
            o_ref[r : r + 1, :] = (
                w0 * xt_ref[0:1, s] + w1 * xt_ref[1:2, s] + w2 * xt_ref[2:3, s] + b
            )

    return _fc_body


def kernel(x, weight, bias):
    B, F = x.shape
    assert F == 3

    b_pad = ((B + _LANES - 1) // _LANES) * _LANES
    if b_pad != B:
        x = jnp.pad(x, ((0, b_pad - B), (0, 0)))
    rows = b_pad // _LANES

    xt = x.T

    wb = jnp.concatenate(
        [weight.reshape(F).astype(jnp.float32), bias.astype(jnp.float32)]
    ).reshape(1, 4)

    rpb = min(_ROWS_PER_BLOCK, rows)
    grid = (pl.cdiv(rows, rpb),)

    out = pl.pallas_call(
        _make_fc_body(rpb),
        out_shape=jax.ShapeDtypeStruct((rows, _LANES), jnp.float32),
        grid=grid,
        in_specs=[
            pl.BlockSpec((3, rpb * _LANES), lambda i: (0, i)),
            pl.BlockSpec(memory_space=pltpu.MemorySpace.SMEM),
        ],
        out_specs=pl.BlockSpec((rpb, _LANES), lambda i: (i, 0)),
        compiler_params=pltpu.CompilerParams(
            dimension_semantics=("parallel",),
        ),
        cost_estimate=pl.CostEstimate(
            flops=6 * b_pad, transcendentals=0, bytes_accessed=16 * b_pad),
    )(xt, wb)

    y = out.reshape(b_pad, 1)
    if b_pad != B:
        y = y[:B]
    return y


# speedup vs baseline: 124.9420x; 1.1768x over previous
import jax
import jax.numpy as jnp
from jax.experimental import pallas as pl
from jax.experimental.pallas import tpu as pltpu

_LANES = 128
_ROWS_PER_BLOCK = 2048


def _make_fc_body(rpb):
    def _fc_body(xt_ref, wb_ref, o_ref):
        w0 = wb_ref[0, 0]
        w1 = wb_ref[0, 1]
        w2 = wb_ref[0, 2]
        b = wb_ref[0, 3]
        for r in range(rpb):
            s = slice(r * _LANES, (r + 1) * _LANES)
            o_ref[r : r + 1, :] = (
                w0 * xt_ref[0:1, s] + w1 * xt_ref[1:2, s] + w2 * xt_ref[2:3, s] + b
            )

    return _fc_body


def kernel(x, weight, bias):
    B, F = x.shape
    assert F == 3

    b_pad = ((B + _LANES - 1) // _LANES) * _LANES
    if b_pad != B:
        x = jnp.pad(x, ((0, b_pad - B), (0, 0)))
    rows = b_pad // _LANES

    xt = x.T

    wb = jnp.concatenate(
        [weight.reshape(F).astype(jnp.float32), bias.astype(jnp.float32)]
    ).reshape(1, 4)

    rpb = min(_ROWS_PER_BLOCK, rows)
    grid = (pl.cdiv(rows, rpb),)

    out = pl.pallas_call(
        _make_fc_body(rpb),
        out_shape=jax.ShapeDtypeStruct((rows, _LANES), jnp.float32),
        grid=grid,
        in_specs=[
            pl.BlockSpec((3, rpb * _LANES), lambda i: (0, i)),
            pl.BlockSpec(memory_space=pltpu.MemorySpace.SMEM),
        ],
        out_specs=pl.BlockSpec((rpb, _LANES), lambda i: (i, 0)),
        compiler_params=pltpu.CompilerParams(
            dimension_semantics=("parallel",),
        ),
        cost_estimate=pl.CostEstimate(
            flops=6 * b_pad, transcendentals=0, bytes_accessed=16 * b_pad),
    )(xt, wb)

    y = out.reshape(b_pad, 1)
    if b_pad != B:
        y = y[:B]
    return y


# speedup vs baseline: 128.7780x; 1.0307x over previous
import jax
import jax.numpy as jnp
from jax.experimental import pallas as pl
from jax.experimental.pallas import tpu as pltpu

_LANES = 128
_ROWS_PER_BLOCK = 4096


def _make_fc_body(rpb):
    def _fc_body(xt_ref, wb_ref, o_ref):
        w0 = wb_ref[0, 0]
        w1 = wb_ref[0, 1]
        w2 = wb_ref[0, 2]
        b = wb_ref[0, 3]
        for r in range(rpb):
            s = slice(r * _LANES, (r + 1) * _LANES)
            o_ref[r : r + 1, :] = (
                w0 * xt_ref[0:1, s] + w1 * xt_ref[1:2, s] + w2 * xt_ref[2:3, s] + b
            )

    return _fc_body


def kernel(x, weight, bias):
    B, F = x.shape
    assert F == 3

    b_pad = ((B + _LANES - 1) // _LANES) * _LANES
    if b_pad != B:
        x = jnp.pad(x, ((0, b_pad - B), (0, 0)))
    rows = b_pad // _LANES

    xt = x.T

    wb = jnp.concatenate(
        [weight.reshape(F).astype(jnp.float32), bias.astype(jnp.float32)]
    ).reshape(1, 4)

    rpb = min(_ROWS_PER_BLOCK, rows)
    grid = (pl.cdiv(rows, rpb),)

    out = pl.pallas_call(
        _make_fc_body(rpb),
        out_shape=jax.ShapeDtypeStruct((rows, _LANES), jnp.float32),
        grid=grid,
        in_specs=[
            pl.BlockSpec((3, rpb * _LANES), lambda i: (0, i)),
            pl.BlockSpec(memory_space=pltpu.MemorySpace.SMEM),
        ],
        out_specs=pl.BlockSpec((rpb, _LANES), lambda i: (i, 0)),
        compiler_params=pltpu.CompilerParams(
            dimension_semantics=("parallel",),
        ),
        cost_estimate=pl.CostEstimate(
            flops=6 * b_pad, transcendentals=0, bytes_accessed=16 * b_pad),
    )(xt, wb)

    y = out.reshape(b_pad, 1)
    if b_pad != B:
        y = y[:B]
    return y


# speedup vs baseline: 166.6883x; 1.2944x over previous
import jax
import jax.numpy as jnp
from jax.experimental import pallas as pl
from jax.experimental.pallas import tpu as pltpu

_LANES = 128
_ROWS_PER_BLOCK = 4096


def _make_fc_body(rpb):
    def _fc_body(xt_ref, wb_ref, o_ref):
        w0 = wb_ref[0, 0]
        w1 = wb_ref[0, 1]
        w2 = wb_ref[0, 2]
        b = wb_ref[0, 3]
        for g in range(rpb // 8):
            s = slice(g * 8 * _LANES, (g + 1) * 8 * _LANES)
            x0 = xt_ref[0, s].reshape(8, _LANES)
            x1 = xt_ref[1, s].reshape(8, _LANES)
            x2 = xt_ref[2, s].reshape(8, _LANES)
            o_ref[g * 8 : (g + 1) * 8, :] = w0 * x0 + w1 * x1 + w2 * x2 + b

    return _fc_body


def kernel(x, weight, bias):
    B, F = x.shape
    assert F == 3

    b_pad = ((B + _LANES - 1) // _LANES) * _LANES
    if b_pad != B:
        x = jnp.pad(x, ((0, b_pad - B), (0, 0)))
    rows = b_pad // _LANES

    xt = x.T

    wb = jnp.concatenate(
        [weight.reshape(F).astype(jnp.float32), bias.astype(jnp.float32)]
    ).reshape(1, 4)

    rpb = min(_ROWS_PER_BLOCK, rows)
    grid = (pl.cdiv(rows, rpb),)

    out = pl.pallas_call(
        _make_fc_body(rpb),
        out_shape=jax.ShapeDtypeStruct((rows, _LANES), jnp.float32),
        grid=grid,
        in_specs=[
            pl.BlockSpec((3, rpb * _LANES), lambda i: (0, i)),
            pl.BlockSpec(memory_space=pltpu.MemorySpace.SMEM),
        ],
        out_specs=pl.BlockSpec((rpb, _LANES), lambda i: (i, 0)),
        compiler_params=pltpu.CompilerParams(
            dimension_semantics=("parallel",),
        ),
        cost_estimate=pl.CostEstimate(
            flops=6 * b_pad, transcendentals=0, bytes_accessed=16 * b_pad),
    )(xt, wb)

    y = out.reshape(b_pad, 1)
    if b_pad != B:
        y = y[:B]
    return y
